# baseline (speedup 1.0000x reference)
import dataclasses
import functools

import jax
import jax.numpy as jnp
from jax import lax
from jax.experimental import pallas as pl
from jax.experimental.pallas import tpu as pltpu
from jax.experimental.pallas import tpu_sc as plsc

NC = 2
NS = 16
NW = NC * NS
LANES = 16


def _vector_mesh():
    return plsc.VectorSubcoreMesh(
        core_axis_name="c", subcore_axis_name="s", num_cores=NC, num_subcores=NS
    )


def _sc_compiler_params():
    cp = pltpu.CompilerParams()
    if "needs_layout_passes" in pltpu.CompilerParams.__dataclass_fields__:
        cp = dataclasses.replace(cp, needs_layout_passes=False)
    return cp


def _scalar_agg(table, src_w, dst_w, n_nodes):
    nw, epw = src_w.shape
    nt = table.shape[0]

    @functools.partial(
        pl.kernel,
        out_type=jax.ShapeDtypeStruct((nw, n_nodes), jnp.float32),
        mesh=_vector_mesh(),
        scratch_types=[
            pltpu.VMEM((nt,), jnp.float32),
            pltpu.VMEM((epw,), jnp.int32),
            pltpu.VMEM((epw,), jnp.int32),
            pltpu.VMEM((n_nodes,), jnp.float32),
        ],
        compiler_params=_sc_compiler_params(),
    )
    def k(t_hbm, s_hbm, d_hbm, out_hbm, t_v, s_v, d_v, acc_v):
        c = lax.axis_index("c")
        s = lax.axis_index("s")
        w = s * NC + c
        pltpu.sync_copy(t_hbm, t_v)
        pltpu.sync_copy(s_hbm.at[w], s_v)
        pltpu.sync_copy(d_hbm.at[w], d_v)

        @pl.loop(0, n_nodes, step=LANES)
        def _(i):
            acc_v[pl.ds(i, LANES)] = jnp.zeros((LANES,), jnp.float32)

        @pl.loop(0, epw, step=LANES)
        def _(i):
            si = s_v[pl.ds(i, LANES)]
            di = d_v[pl.ds(i, LANES)]
            vals = plsc.load_gather(t_v, [si])
            plsc.addupdate_scatter(acc_v, [di], vals)

        pltpu.sync_copy(acc_v, out_hbm.at[w])

    return k(table, src_w, dst_w)


def _row_agg(g, src_p, dst_p, zeros_tile, n_pad):
    n, d = g.shape
    nw, n_chunks, kk = src_p.shape
    rpt = n_pad // NS

    half = n_chunks // 2
    assert n_chunks % 2 == 0

    @functools.partial(
        pl.kernel,
        out_type=jax.ShapeDtypeStruct((NC, n_pad, d), jnp.float32),
        mesh=_vector_mesh(),
        scratch_types=[
            pltpu.VMEM((half, kk), jnp.int32),
            pltpu.VMEM((half, kk), jnp.int32),
            pltpu.VMEM((2, kk, d), jnp.float32),
            pltpu.VMEM_SHARED((n_pad, d), jnp.float32),
            pltpu.SemaphoreType.DMA,
            pltpu.SemaphoreType.DMA,
        ],
        compiler_params=_sc_compiler_params(),
    )
    def k(g_hbm, si_hbm, di_hbm, z_hbm, out_hbm, si_v, di_v, rows_v, acc_sh,
          gsem0, gsem1):
        c = lax.axis_index("c")
        s = lax.axis_index("s")
        w = s * NC + c
        pltpu.sync_copy(z_hbm, acc_sh.at[pl.ds(s * rpt, rpt)])
        plsc.subcore_barrier()

        def issue(j, buf, sem):
            pltpu.async_copy(g_hbm.at[si_v.at[j]], rows_v.at[buf], sem)

        def wait(buf, sem):
            pltpu.make_async_copy(g_hbm.at[pl.ds(0, kk)], rows_v.at[buf], sem).wait()

        def phase(p):
            pltpu.sync_copy(si_hbm.at[w, pl.ds(p * half, half)], si_v)
            pltpu.sync_copy(di_hbm.at[w, pl.ds(p * half, half)], di_v)
            @pl.loop(0, half)
            def _(j):
                pltpu.sync_copy(g_hbm.at[si_v.at[j]], rows_v.at[0])

        phase(0)
        phase(1)

        plsc.subcore_barrier()
        pltpu.sync_copy(
            acc_sh.at[pl.ds(s * rpt, rpt)],
            out_hbm.at[c, pl.ds(s * rpt, rpt)],
        )

    return k(g, src_p, dst_p, zeros_tile)


def _tc_prep(deg_part, x, w1):
    n, din = x.shape
    dh = w1.shape[1]
    r = 1024
    grid = -(-n // r)

    def body(dp_ref, x_ref, w_ref, g_ref, dis_ref):
        deg = jnp.sum(dp_ref[...], axis=0, keepdims=True) + 1.0
        dis = lax.rsqrt(deg)
        dis_ref[...] = dis
        h = jnp.dot(x_ref[...], w_ref[...], preferred_element_type=jnp.float32)
        g_ref[...] = h * jnp.transpose(dis)

    return pl.pallas_call(
        body,
        grid=(grid,),
        in_specs=[
            pl.BlockSpec((deg_part.shape[0], r), lambda i: (0, i)),
            pl.BlockSpec((r, din), lambda i: (i, 0)),
            pl.BlockSpec((din, dh), lambda i: (0, 0)),
        ],
        out_specs=[
            pl.BlockSpec((r, dh), lambda i: (i, 0)),
            pl.BlockSpec((1, r), lambda i: (0, i)),
        ],
        out_shape=[
            jax.ShapeDtypeStruct((n, dh), jnp.float32),
            jax.ShapeDtypeStruct((1, n), jnp.float32),
        ],
    )(deg_part, x, w1)


def _tc_layer2(acc, g, dis, b1_row, w2_row):
    n, dh = g.shape
    r = 1024
    grid = -(-n // r)

    def body(acc_ref, g_ref, dis_ref, b1_ref, w2_ref, g2_ref):
        ssum = acc_ref[0] + acc_ref[1] + g_ref[...]
        dis_c = jnp.transpose(dis_ref[...])
        out1 = ssum * dis_c + b1_ref[...]
        relu = jnp.maximum(out1, 0.0)
        v = jnp.sum(relu * w2_ref[...], axis=1, keepdims=True)
        g2_ref[...] = dis_c * v

    return pl.pallas_call(
        body,
        grid=(grid,),
        in_specs=[
            pl.BlockSpec((NC, r, dh), lambda i: (0, i, 0)),
            pl.BlockSpec((r, dh), lambda i: (i, 0)),
            pl.BlockSpec((1, r), lambda i: (0, i)),
            pl.BlockSpec((1, dh), lambda i: (0, 0)),
            pl.BlockSpec((1, dh), lambda i: (0, 0)),
        ],
        out_specs=pl.BlockSpec((r, 1), lambda i: (i, 0)),
        out_shape=jax.ShapeDtypeStruct((n, 1), jnp.float32),
    )(acc, g, dis, b1_row, w2_row)


def _tc_final(acc2_part, g2_row, dis, b2_val):
    nw, n = acc2_part.shape
    r = 1024
    grid = -(-n // r)

    def body(a2_ref, g2_ref, dis_ref, b2_ref, o_ref):
        tot = jnp.sum(a2_ref[...], axis=0, keepdims=True) + g2_ref[...]
        o_ref[...] = dis_ref[...] * tot + b2_ref[0, 0]

    return pl.pallas_call(
        body,
        grid=(grid,),
        in_specs=[
            pl.BlockSpec((nw, r), lambda i: (0, i)),
            pl.BlockSpec((1, r), lambda i: (0, i)),
            pl.BlockSpec((1, r), lambda i: (0, i)),
            pl.BlockSpec((1, 1), lambda i: (0, 0)),
        ],
        out_specs=pl.BlockSpec((1, r), lambda i: (0, i)),
        out_shape=jax.ShapeDtypeStruct((1, n), jnp.float32),
    )(acc2_part, g2_row, dis, b2_val)


def kernel(x, edge_index, W1, b1, W2, b2):
    n, din = x.shape
    dh = W1.shape[1]
    e = edge_index.shape[1]
    src = edge_index[0].astype(jnp.int32)
    dst = edge_index[1].astype(jnp.int32)

    epw = e // NW
    src_w = src.reshape(NW, epw)
    dst_w = dst.reshape(NW, epw)

    ones_t = jnp.ones((n,), jnp.float32)
    deg_part = _scalar_agg(ones_t, dst_w, dst_w, n)

    g, dis = _tc_prep(deg_part, x, W1)

    kk = 128
    n_chunks = -(-e // (NW * kk))
    n_chunks += n_chunks % 2
    e_pad = NW * kk * n_chunks
    pad = e_pad - e
    src_p = jnp.concatenate([src, jnp.zeros((pad,), jnp.int32)]).reshape(
        NW, n_chunks, kk
    )
    dst_p = jnp.concatenate([dst, jnp.full((pad,), n, jnp.int32)]).reshape(
        NW, n_chunks, kk
    )
    n_pad = -(-(n + 1) // (NS * 8)) * (NS * 8)
    zeros_tile = jnp.zeros((n_pad // NS, dh), jnp.float32)
    acc = _row_agg(g, src_p, dst_p, zeros_tile, n_pad)
    acc = acc[:, :n, :]

    g2_col = _tc_layer2(acc, g, dis, b1.reshape(1, dh), W2.reshape(1, dh))
    g2 = g2_col.reshape(n)

    acc2_part = _scalar_agg(g2, src_w, dst_w, n)

    out = _tc_final(acc2_part, g2.reshape(1, n), dis, b2.reshape(1, 1))
    return out.reshape(n)


# speedup vs baseline: 1.0351x; 1.0351x over previous
import dataclasses
import functools

import jax
import jax.numpy as jnp
from jax import lax
from jax.experimental import pallas as pl
from jax.experimental.pallas import tpu as pltpu
from jax.experimental.pallas import tpu_sc as plsc

NC = 2
NS = 16
NW = NC * NS
LANES = 16


def _vector_mesh():
    return plsc.VectorSubcoreMesh(
        core_axis_name="c", subcore_axis_name="s", num_cores=NC, num_subcores=NS
    )


def _sc_compiler_params():
    cp = pltpu.CompilerParams()
    if "needs_layout_passes" in pltpu.CompilerParams.__dataclass_fields__:
        cp = dataclasses.replace(cp, needs_layout_passes=False)
    return cp


def _scalar_agg(table, src_w, dst_w, n_nodes):
    nw, epw = src_w.shape
    nt = table.shape[0]

    @functools.partial(
        pl.kernel,
        out_type=jax.ShapeDtypeStruct((nw, n_nodes), jnp.float32),
        mesh=_vector_mesh(),
        scratch_types=[
            pltpu.VMEM((nt,), jnp.float32),
            pltpu.VMEM((epw,), jnp.int32),
            pltpu.VMEM((epw,), jnp.int32),
            pltpu.VMEM((n_nodes,), jnp.float32),
        ],
        compiler_params=_sc_compiler_params(),
    )
    def k(t_hbm, s_hbm, d_hbm, out_hbm, t_v, s_v, d_v, acc_v):
        c = lax.axis_index("c")
        s = lax.axis_index("s")
        w = s * NC + c
        pltpu.sync_copy(t_hbm, t_v)
        pltpu.sync_copy(s_hbm.at[w], s_v)
        pltpu.sync_copy(d_hbm.at[w], d_v)

        @pl.loop(0, n_nodes, step=LANES)
        def _(i):
            acc_v[pl.ds(i, LANES)] = jnp.zeros((LANES,), jnp.float32)

        @pl.loop(0, epw, step=LANES)
        def _(i):
            si = s_v[pl.ds(i, LANES)]
            di = d_v[pl.ds(i, LANES)]
            vals = plsc.load_gather(t_v, [si])
            plsc.addupdate_scatter(acc_v, [di], vals)

        pltpu.sync_copy(acc_v, out_hbm.at[w])

    return k(table, src_w, dst_w)


def _row_agg(g, src_p, dst_p, zeros_tile, n_pad):
    n, d = g.shape
    nw, n_chunks, kk = src_p.shape
    rpt = n_pad // NS

    @functools.partial(
        pl.kernel,
        out_type=jax.ShapeDtypeStruct((NC, n_pad, d), jnp.float32),
        mesh=_vector_mesh(),
        scratch_types=[
            pltpu.VMEM((n_chunks // 2, kk), jnp.int32),
            pltpu.VMEM((n_chunks // 2, kk), jnp.int32),
            pltpu.VMEM((kk, d), jnp.float32),
            pltpu.VMEM((kk, d), jnp.float32),
            pltpu.VMEM_SHARED((n_pad, d), jnp.float32),
            pltpu.SemaphoreType.DMA,
            pltpu.SemaphoreType.DMA,
        ],
        compiler_params=_sc_compiler_params(),
    )
    def k(g_hbm, si_hbm, di_hbm, z_hbm, out_hbm, si_v, di_v, ra, rb, acc_sh,
          sa, sb):
        half = n_chunks // 2
        c = lax.axis_index("c")
        s = lax.axis_index("s")
        w = s * NC + c
        pltpu.sync_copy(z_hbm, ra)
        base = s * rpt
        off = 0
        while off < rpt:
            sz = min(kk, rpt - off)
            pltpu.sync_copy(ra.at[pl.ds(0, sz)], acc_sh.at[pl.ds(base + off, sz)])
            off += sz
        plsc.subcore_barrier()

        def issue(j, buf, sem):
            pltpu.async_copy(g_hbm.at[si_v.at[j]], buf, sem)

        def wait(buf, sem):
            pltpu.make_async_copy(g_hbm.at[pl.ds(0, kk)], buf, sem).wait()

        def phase(p):
            pltpu.sync_copy(si_hbm.at[w, pl.ds(p * half, half)], si_v)
            pltpu.sync_copy(di_hbm.at[w, pl.ds(p * half, half)], di_v)
            issue(0, ra, sa)
            issue(1, rb, sb)

            @pl.loop(0, half, step=2)
            def _(j):
                wait(ra, sa)
                pltpu.sync_copy(ra, acc_sh.at[di_v.at[j]], add=True)

                @pl.when(j + 2 < half)
                def _():
                    issue(j + 2, ra, sa)

                wait(rb, sb)
                pltpu.sync_copy(rb, acc_sh.at[di_v.at[j + 1]], add=True)

                @pl.when(j + 3 < half)
                def _():
                    issue(j + 3, rb, sb)

        phase(0)
        phase(1)

        plsc.subcore_barrier()
        pltpu.sync_copy(
            acc_sh.at[pl.ds(s * rpt, rpt)],
            out_hbm.at[c, pl.ds(s * rpt, rpt)],
        )

    return k(g, src_p, dst_p, zeros_tile)


def _tc_prep(deg_part, x, w1):
    n, din = x.shape
    dh = w1.shape[1]
    r = 1024
    grid = -(-n // r)

    def body(dp_ref, x_ref, w_ref, g_ref, dis_ref):
        deg = jnp.sum(dp_ref[...], axis=0, keepdims=True) + 1.0
        dis = lax.rsqrt(deg)
        dis_ref[...] = dis
        h = jnp.dot(x_ref[...], w_ref[...], preferred_element_type=jnp.float32)
        g_ref[...] = h * jnp.transpose(dis)

    return pl.pallas_call(
        body,
        grid=(grid,),
        in_specs=[
            pl.BlockSpec((deg_part.shape[0], r), lambda i: (0, i)),
            pl.BlockSpec((r, din), lambda i: (i, 0)),
            pl.BlockSpec((din, dh), lambda i: (0, 0)),
        ],
        out_specs=[
            pl.BlockSpec((r, dh), lambda i: (i, 0)),
            pl.BlockSpec((1, r), lambda i: (0, i)),
        ],
        out_shape=[
            jax.ShapeDtypeStruct((n, dh), jnp.float32),
            jax.ShapeDtypeStruct((1, n), jnp.float32),
        ],
    )(deg_part, x, w1)


def _tc_layer2(acc, g, dis, b1_row, w2_row):
    n, dh = g.shape
    r = 1024
    grid = -(-n // r)

    def body(acc_ref, g_ref, dis_ref, b1_ref, w2_ref, g2_ref):
        ssum = acc_ref[0] + acc_ref[1] + g_ref[...]
        dis_c = jnp.transpose(dis_ref[...])
        out1 = ssum * dis_c + b1_ref[...]
        relu = jnp.maximum(out1, 0.0)
        v = jnp.sum(relu * w2_ref[...], axis=1, keepdims=True)
        g2_ref[...] = dis_c * v

    return pl.pallas_call(
        body,
        grid=(grid,),
        in_specs=[
            pl.BlockSpec((NC, r, dh), lambda i: (0, i, 0)),
            pl.BlockSpec((r, dh), lambda i: (i, 0)),
            pl.BlockSpec((1, r), lambda i: (0, i)),
            pl.BlockSpec((1, dh), lambda i: (0, 0)),
            pl.BlockSpec((1, dh), lambda i: (0, 0)),
        ],
        out_specs=pl.BlockSpec((r, 1), lambda i: (i, 0)),
        out_shape=jax.ShapeDtypeStruct((n, 1), jnp.float32),
    )(acc, g, dis, b1_row, w2_row)


def _tc_final(acc2_part, g2_row, dis, b2_val):
    nw, n = acc2_part.shape
    r = 1024
    grid = -(-n // r)

    def body(a2_ref, g2_ref, dis_ref, b2_ref, o_ref):
        tot = jnp.sum(a2_ref[...], axis=0, keepdims=True) + g2_ref[...]
        o_ref[...] = dis_ref[...] * tot + b2_ref[0, 0]

    return pl.pallas_call(
        body,
        grid=(grid,),
        in_specs=[
            pl.BlockSpec((nw, r), lambda i: (0, i)),
            pl.BlockSpec((1, r), lambda i: (0, i)),
            pl.BlockSpec((1, r), lambda i: (0, i)),
            pl.BlockSpec((1, 1), lambda i: (0, 0)),
        ],
        out_specs=pl.BlockSpec((1, r), lambda i: (0, i)),
        out_shape=jax.ShapeDtypeStruct((1, n), jnp.float32),
    )(acc2_part, g2_row, dis, b2_val)


def kernel(x, edge_index, W1, b1, W2, b2):
    n, din = x.shape
    dh = W1.shape[1]
    e = edge_index.shape[1]
    src = edge_index[0].astype(jnp.int32)
    dst = edge_index[1].astype(jnp.int32)

    epw = e // NW
    src_w = src.reshape(NW, epw)
    dst_w = dst.reshape(NW, epw)

    ones_t = jnp.ones((n,), jnp.float32)
    deg_part = _scalar_agg(ones_t, dst_w, dst_w, n)

    g, dis = _tc_prep(deg_part, x, W1)

    kk = 128
    n_chunks = -(-e // (NW * kk))
    n_chunks = -(-n_chunks // 4) * 4
    e_pad = NW * kk * n_chunks
    pad = e_pad - e
    src_p = jnp.concatenate([src, jnp.zeros((pad,), jnp.int32)]).reshape(
        NW, n_chunks, kk
    )
    dst_p = jnp.concatenate([dst, jnp.full((pad,), n, jnp.int32)]).reshape(
        NW, n_chunks, kk
    )
    n_pad = -(-(n + 1) // (NS * 8)) * (NS * 8)
    zeros_tile = jnp.zeros((kk, dh), jnp.float32)
    acc = _row_agg(g, src_p, dst_p, zeros_tile, n_pad)
    acc = acc[:, :n, :]

    g2_col = _tc_layer2(acc, g, dis, b1.reshape(1, dh), W2.reshape(1, dh))
    g2 = g2_col.reshape(n)

    acc2_part = _scalar_agg(g2, src_w, dst_w, n)

    out = _tc_final(acc2_part, g2.reshape(1, n), dis, b2.reshape(1, 1))
    return out.reshape(n)


# speedup vs baseline: 2.0126x; 1.9443x over previous
import dataclasses
import functools

import jax
import jax.numpy as jnp
from jax import lax
from jax.experimental import pallas as pl
from jax.experimental.pallas import tpu as pltpu
from jax.experimental.pallas import tpu_sc as plsc

NC = 2
NS = 16
NW = NC * NS
LANES = 16


def _vector_mesh():
    return plsc.VectorSubcoreMesh(
        core_axis_name="c", subcore_axis_name="s", num_cores=NC, num_subcores=NS
    )


def _sc_compiler_params():
    cp = pltpu.CompilerParams()
    if "needs_layout_passes" in pltpu.CompilerParams.__dataclass_fields__:
        cp = dataclasses.replace(cp, needs_layout_passes=False)
    return cp


def _scalar_agg(table, src_w, dst_w, n_nodes):
    nw, epw = src_w.shape
    nt = table.shape[0]

    @functools.partial(
        pl.kernel,
        out_type=jax.ShapeDtypeStruct((nw, n_nodes), jnp.float32),
        mesh=_vector_mesh(),
        scratch_types=[
            pltpu.VMEM((nt,), jnp.float32),
            pltpu.VMEM((epw,), jnp.int32),
            pltpu.VMEM((epw,), jnp.int32),
            pltpu.VMEM((n_nodes,), jnp.float32),
        ],
        compiler_params=_sc_compiler_params(),
    )
    def k(t_hbm, s_hbm, d_hbm, out_hbm, t_v, s_v, d_v, acc_v):
        c = lax.axis_index("c")
        s = lax.axis_index("s")
        w = s * NC + c
        pltpu.sync_copy(t_hbm, t_v)
        pltpu.sync_copy(s_hbm.at[w], s_v)
        pltpu.sync_copy(d_hbm.at[w], d_v)

        @pl.loop(0, n_nodes, step=LANES)
        def _(i):
            acc_v[pl.ds(i, LANES)] = jnp.zeros((LANES,), jnp.float32)

        @pl.loop(0, epw, step=LANES)
        def _(i):
            si = s_v[pl.ds(i, LANES)]
            di = d_v[pl.ds(i, LANES)]
            vals = plsc.load_gather(t_v, [si])
            plsc.addupdate_scatter(acc_v, [di], vals)

        pltpu.sync_copy(acc_v, out_hbm.at[w])

    return k(table, src_w, dst_w)


def _row_agg(g, src_p, dst_p, zeros_tile, n_pad):
    n, d = g.shape
    nw, n_chunks, kk = src_p.shape
    rpt = n_pad // NS

    @functools.partial(
        pl.kernel,
        out_type=jax.ShapeDtypeStruct((NC, n_pad, d), jnp.float32),
        mesh=_vector_mesh(),
        scratch_types=[
            pltpu.VMEM((n_chunks, kk), jnp.int32),
            pltpu.VMEM((n_chunks, kk), jnp.int32),
            pltpu.VMEM((kk, d), jnp.float32),
            pltpu.VMEM_SHARED((n_pad, d), jnp.float32),
        ],
        compiler_params=_sc_compiler_params(),
    )
    def k(g_hbm, si_hbm, di_hbm, z_hbm, out_hbm, si_v, di_v, rows_v, acc_sh):
        c = lax.axis_index("c")
        s = lax.axis_index("s")
        w = s * NC + c
        pltpu.sync_copy(z_hbm, rows_v)
        base = s * rpt
        off = 0
        while off < rpt:
            sz = min(kk, rpt - off)
            pltpu.sync_copy(rows_v.at[pl.ds(0, sz)], acc_sh.at[pl.ds(base + off, sz)])
            off += sz
        pltpu.sync_copy(si_hbm.at[w], si_v)
        pltpu.sync_copy(di_hbm.at[w], di_v)
        plsc.subcore_barrier()

        @pl.loop(0, n_chunks)
        def _(j):
            pltpu.sync_copy(g_hbm.at[si_v.at[j]], rows_v)
            pltpu.sync_copy(rows_v, acc_sh.at[di_v.at[j]], add=True)

        plsc.subcore_barrier()
        pltpu.sync_copy(
            acc_sh.at[pl.ds(s * rpt, rpt)],
            out_hbm.at[c, pl.ds(s * rpt, rpt)],
        )

    return k(g, src_p, dst_p, zeros_tile)


def _tc_prep(deg_part, x, w1):
    n, din = x.shape
    dh = w1.shape[1]
    r = 1024
    grid = -(-n // r)

    def body(dp_ref, x_ref, w_ref, g_ref, dis_ref):
        deg = jnp.sum(dp_ref[...], axis=0, keepdims=True) + 1.0
        dis = lax.rsqrt(deg)
        dis_ref[...] = dis
        h = jnp.dot(x_ref[...], w_ref[...], preferred_element_type=jnp.float32)
        g_ref[...] = h * jnp.transpose(dis)

    return pl.pallas_call(
        body,
        grid=(grid,),
        in_specs=[
            pl.BlockSpec((deg_part.shape[0], r), lambda i: (0, i)),
            pl.BlockSpec((r, din), lambda i: (i, 0)),
            pl.BlockSpec((din, dh), lambda i: (0, 0)),
        ],
        out_specs=[
            pl.BlockSpec((r, dh), lambda i: (i, 0)),
            pl.BlockSpec((1, r), lambda i: (0, i)),
        ],
        out_shape=[
            jax.ShapeDtypeStruct((n, dh), jnp.float32),
            jax.ShapeDtypeStruct((1, n), jnp.float32),
        ],
    )(deg_part, x, w1)


def _tc_layer2(acc, g, dis, b1_row, w2_row):
    n, dh = g.shape
    r = 1024
    grid = -(-n // r)

    def body(acc_ref, g_ref, dis_ref, b1_ref, w2_ref, g2_ref):
        ssum = acc_ref[0] + acc_ref[1] + g_ref[...]
        dis_c = jnp.transpose(dis_ref[...])
        out1 = ssum * dis_c + b1_ref[...]
        relu = jnp.maximum(out1, 0.0)
        v = jnp.sum(relu * w2_ref[...], axis=1, keepdims=True)
        g2_ref[...] = dis_c * v

    return pl.pallas_call(
        body,
        grid=(grid,),
        in_specs=[
            pl.BlockSpec((NC, r, dh), lambda i: (0, i, 0)),
            pl.BlockSpec((r, dh), lambda i: (i, 0)),
            pl.BlockSpec((1, r), lambda i: (0, i)),
            pl.BlockSpec((1, dh), lambda i: (0, 0)),
            pl.BlockSpec((1, dh), lambda i: (0, 0)),
        ],
        out_specs=pl.BlockSpec((r, 1), lambda i: (i, 0)),
        out_shape=jax.ShapeDtypeStruct((n, 1), jnp.float32),
    )(acc, g, dis, b1_row, w2_row)


def _tc_final(acc2_part, g2_row, dis, b2_val):
    nw, n = acc2_part.shape
    r = 1024
    grid = -(-n // r)

    def body(a2_ref, g2_ref, dis_ref, b2_ref, o_ref):
        tot = jnp.sum(a2_ref[...], axis=0, keepdims=True) + g2_ref[...]
        o_ref[...] = dis_ref[...] * tot + b2_ref[0, 0]

    return pl.pallas_call(
        body,
        grid=(grid,),
        in_specs=[
            pl.BlockSpec((nw, r), lambda i: (0, i)),
            pl.BlockSpec((1, r), lambda i: (0, i)),
            pl.BlockSpec((1, r), lambda i: (0, i)),
            pl.BlockSpec((1, 1), lambda i: (0, 0)),
        ],
        out_specs=pl.BlockSpec((1, r), lambda i: (0, i)),
        out_shape=jax.ShapeDtypeStruct((1, n), jnp.float32),
    )(acc2_part, g2_row, dis, b2_val)


def kernel(x, edge_index, W1, b1, W2, b2):
    n, din = x.shape
    dh = W1.shape[1]
    e = edge_index.shape[1]
    src = edge_index[0].astype(jnp.int32)
    dst = edge_index[1].astype(jnp.int32)

    epw = e // NW
    src_w = src.reshape(NW, epw)
    dst_w = dst.reshape(NW, epw)

    ones_t = jnp.ones((n,), jnp.float32)
    deg_part = _scalar_agg(ones_t, dst_w, dst_w, n)

    g, dis = _tc_prep(deg_part, x, W1)

    kk = 128
    n_chunks = -(-e // (NW * kk))
    n_chunks = -(-n_chunks // 4) * 4
    e_pad = NW * kk * n_chunks
    pad = e_pad - e
    n_pad_probe = -(-(n + 1) // (NS * 8)) * (NS * 8)
    pad_iota = jnp.arange(pad, dtype=jnp.int32)
    src_p = jnp.concatenate([src, pad_iota % n]).reshape(NW, n_chunks, kk)
    dst_p = jnp.concatenate(
        [dst, n + pad_iota % (n_pad_probe - n)]
    ).reshape(NW, n_chunks, kk)
    n_pad = -(-(n + 1) // (NS * 8)) * (NS * 8)
    zeros_tile = jnp.zeros((kk, dh), jnp.float32)
    acc = _row_agg(g, src_p, dst_p, zeros_tile, n_pad)
    acc = acc[:, :n, :]

    g2_col = _tc_layer2(acc, g, dis, b1.reshape(1, dh), W2.reshape(1, dh))
    g2 = g2_col.reshape(n)

    acc2_part = _scalar_agg(g2, src_w, dst_w, n)

    out = _tc_final(acc2_part, g2.reshape(1, n), dis, b2.reshape(1, 1))
    return out.reshape(n)


# speedup vs baseline: 2.6172x; 1.3004x over previous
import dataclasses
import functools

import jax
import jax.numpy as jnp
from jax import lax
from jax.experimental import pallas as pl
from jax.experimental.pallas import tpu as pltpu
from jax.experimental.pallas import tpu_sc as plsc

NC = 2
NS = 16
NW = NC * NS
LANES = 16


def _vector_mesh():
    return plsc.VectorSubcoreMesh(
        core_axis_name="c", subcore_axis_name="s", num_cores=NC, num_subcores=NS
    )


def _sc_compiler_params():
    cp = pltpu.CompilerParams()
    if "needs_layout_passes" in pltpu.CompilerParams.__dataclass_fields__:
        cp = dataclasses.replace(cp, needs_layout_passes=False)
    return cp


def _scalar_agg(table, src_w, dst_w, n_nodes):
    nw, epw = src_w.shape
    nt = table.shape[0]

    @functools.partial(
        pl.kernel,
        out_type=jax.ShapeDtypeStruct((nw, n_nodes), jnp.float32),
        mesh=_vector_mesh(),
        scratch_types=[
            pltpu.VMEM((nt,), jnp.float32),
            pltpu.VMEM((epw,), jnp.int32),
            pltpu.VMEM((epw,), jnp.int32),
            pltpu.VMEM((n_nodes,), jnp.float32),
        ],
        compiler_params=_sc_compiler_params(),
    )
    def k(t_hbm, s_hbm, d_hbm, out_hbm, t_v, s_v, d_v, acc_v):
        c = lax.axis_index("c")
        s = lax.axis_index("s")
        w = s * NC + c
        pltpu.sync_copy(t_hbm, t_v)
        pltpu.sync_copy(s_hbm.at[w], s_v)
        pltpu.sync_copy(d_hbm.at[w], d_v)

        @pl.loop(0, n_nodes, step=LANES)
        def _(i):
            acc_v[pl.ds(i, LANES)] = jnp.zeros((LANES,), jnp.float32)

        @pl.loop(0, epw, step=LANES)
        def _(i):
            si = s_v[pl.ds(i, LANES)]
            di = d_v[pl.ds(i, LANES)]
            vals = plsc.load_gather(t_v, [si])
            plsc.addupdate_scatter(acc_v, [di], vals)

        pltpu.sync_copy(acc_v, out_hbm.at[w])

    return k(table, src_w, dst_w)


def _row_agg(g, src_p, dst_p, zeros_tile, n_pad):
    n, d = g.shape
    nw, n_chunks, kk = src_p.shape
    rpt = n_pad // NS

    @functools.partial(
        pl.kernel,
        out_type=jax.ShapeDtypeStruct((NC, n_pad, d), jnp.float32),
        mesh=_vector_mesh(),
        scratch_types=[
            pltpu.VMEM((n_chunks // 2, kk), jnp.int32),
            pltpu.VMEM((n_chunks // 2, kk), jnp.int32),
            pltpu.VMEM((kk, d), jnp.float32),
            pltpu.VMEM((kk, d), jnp.float32),
            pltpu.VMEM_SHARED((n_pad, d), jnp.float32),
            pltpu.SemaphoreType.DMA,
            pltpu.SemaphoreType.DMA,
        ],
        compiler_params=_sc_compiler_params(),
    )
    def k(g_hbm, si_hbm, di_hbm, z_hbm, out_hbm, si_v, di_v, ra, rb, acc_sh,
          sa, sb):
        half = n_chunks // 2
        c = lax.axis_index("c")
        s = lax.axis_index("s")
        w = s * NC + c
        pltpu.sync_copy(z_hbm, ra)
        base = s * rpt
        off = 0
        while off < rpt:
            sz = min(kk, rpt - off)
            pltpu.sync_copy(ra.at[pl.ds(0, sz)], acc_sh.at[pl.ds(base + off, sz)])
            off += sz
        plsc.subcore_barrier()

        def issue(j, buf, sem):
            pltpu.async_copy(g_hbm.at[si_v.at[j]], buf, sem)

        def wait(buf, sem):
            pltpu.make_async_copy(g_hbm.at[pl.ds(0, kk)], buf, sem).wait()

        def phase(p):
            pltpu.sync_copy(si_hbm.at[w, pl.ds(p * half, half)], si_v)
            pltpu.sync_copy(di_hbm.at[w, pl.ds(p * half, half)], di_v)
            issue(0, ra, sa)
            issue(1, rb, sb)

            @pl.loop(0, half, step=2)
            def _(j):
                wait(ra, sa)
                pltpu.sync_copy(ra, acc_sh.at[di_v.at[j]], add=True)

                @pl.when(j + 2 < half)
                def _():
                    issue(j + 2, ra, sa)

                wait(rb, sb)
                pltpu.sync_copy(rb, acc_sh.at[di_v.at[j + 1]], add=True)

                @pl.when(j + 3 < half)
                def _():
                    issue(j + 3, rb, sb)

        phase(0)
        phase(1)

        plsc.subcore_barrier()
        pltpu.sync_copy(
            acc_sh.at[pl.ds(s * rpt, rpt)],
            out_hbm.at[c, pl.ds(s * rpt, rpt)],
        )

    return k(g, src_p, dst_p, zeros_tile)


def _tc_prep(deg_part, x, w1):
    n, din = x.shape
    dh = w1.shape[1]
    r = 1024
    grid = -(-n // r)

    def body(dp_ref, x_ref, w_ref, g_ref, dis_ref):
        deg = jnp.sum(dp_ref[...], axis=0, keepdims=True) + 1.0
        dis = lax.rsqrt(deg)
        dis_ref[...] = dis
        h = jnp.dot(x_ref[...], w_ref[...], preferred_element_type=jnp.float32)
        g_ref[...] = h * jnp.transpose(dis)

    return pl.pallas_call(
        body,
        grid=(grid,),
        in_specs=[
            pl.BlockSpec((deg_part.shape[0], r), lambda i: (0, i)),
            pl.BlockSpec((r, din), lambda i: (i, 0)),
            pl.BlockSpec((din, dh), lambda i: (0, 0)),
        ],
        out_specs=[
            pl.BlockSpec((r, dh), lambda i: (i, 0)),
            pl.BlockSpec((1, r), lambda i: (0, i)),
        ],
        out_shape=[
            jax.ShapeDtypeStruct((n, dh), jnp.float32),
            jax.ShapeDtypeStruct((1, n), jnp.float32),
        ],
    )(deg_part, x, w1)


def _tc_layer2(acc, g, dis, b1_row, w2_row):
    n, dh = g.shape
    r = 1024
    grid = -(-n // r)

    def body(acc_ref, g_ref, dis_ref, b1_ref, w2_ref, g2_ref):
        ssum = acc_ref[0] + acc_ref[1] + g_ref[...]
        dis_c = jnp.transpose(dis_ref[...])
        out1 = ssum * dis_c + b1_ref[...]
        relu = jnp.maximum(out1, 0.0)
        v = jnp.sum(relu * w2_ref[...], axis=1, keepdims=True)
        g2_ref[...] = dis_c * v

    return pl.pallas_call(
        body,
        grid=(grid,),
        in_specs=[
            pl.BlockSpec((NC, r, dh), lambda i: (0, i, 0)),
            pl.BlockSpec((r, dh), lambda i: (i, 0)),
            pl.BlockSpec((1, r), lambda i: (0, i)),
            pl.BlockSpec((1, dh), lambda i: (0, 0)),
            pl.BlockSpec((1, dh), lambda i: (0, 0)),
        ],
        out_specs=pl.BlockSpec((r, 1), lambda i: (i, 0)),
        out_shape=jax.ShapeDtypeStruct((n, 1), jnp.float32),
    )(acc, g, dis, b1_row, w2_row)


def _tc_final(acc2_part, g2_row, dis, b2_val):
    nw, n = acc2_part.shape
    r = 1024
    grid = -(-n // r)

    def body(a2_ref, g2_ref, dis_ref, b2_ref, o_ref):
        tot = jnp.sum(a2_ref[...], axis=0, keepdims=True) + g2_ref[...]
        o_ref[...] = dis_ref[...] * tot + b2_ref[0, 0]

    return pl.pallas_call(
        body,
        grid=(grid,),
        in_specs=[
            pl.BlockSpec((nw, r), lambda i: (0, i)),
            pl.BlockSpec((1, r), lambda i: (0, i)),
            pl.BlockSpec((1, r), lambda i: (0, i)),
            pl.BlockSpec((1, 1), lambda i: (0, 0)),
        ],
        out_specs=pl.BlockSpec((1, r), lambda i: (0, i)),
        out_shape=jax.ShapeDtypeStruct((1, n), jnp.float32),
    )(acc2_part, g2_row, dis, b2_val)


def kernel(x, edge_index, W1, b1, W2, b2):
    n, din = x.shape
    dh = W1.shape[1]
    e = edge_index.shape[1]
    src = edge_index[0].astype(jnp.int32)
    dst = edge_index[1].astype(jnp.int32)

    epw = e // NW
    src_w = src.reshape(NW, epw)
    dst_w = dst.reshape(NW, epw)

    ones_t = jnp.ones((n,), jnp.float32)
    deg_part = _scalar_agg(ones_t, dst_w, dst_w, n)

    g, dis = _tc_prep(deg_part, x, W1)

    kk = 128
    n_chunks = -(-e // (NW * kk))
    n_chunks = -(-n_chunks // 4) * 4
    e_pad = NW * kk * n_chunks
    pad = e_pad - e
    n_pad_probe = -(-(n + 1) // (NS * 8)) * (NS * 8)
    pad_iota = jnp.arange(pad, dtype=jnp.int32)
    src_p = jnp.concatenate([src, pad_iota % n]).reshape(NW, n_chunks, kk)
    dst_p = jnp.concatenate(
        [dst, n + pad_iota % (n_pad_probe - n)]
    ).reshape(NW, n_chunks, kk)
    n_pad = -(-(n + 1) // (NS * 8)) * (NS * 8)
    zeros_tile = jnp.zeros((kk, dh), jnp.float32)
    acc = _row_agg(g, src_p, dst_p, zeros_tile, n_pad)
    acc = acc[:, :n, :]

    g2_col = _tc_layer2(acc, g, dis, b1.reshape(1, dh), W2.reshape(1, dh))
    g2 = g2_col.reshape(n)

    acc2_part = _scalar_agg(g2, src_w, dst_w, n)

    out = _tc_final(acc2_part, g2.reshape(1, n), dis, b2.reshape(1, 1))
    return out.reshape(n)
